# baseline (speedup 1.0000x reference)
import functools

import jax
import jax.numpy as jnp
from jax import lax
from jax.experimental import pallas as pl
from jax.experimental.pallas import tpu as pltpu
from jax.experimental.pallas import tpu_sc as plsc

_NUM_OPT = 4
_NUM_D = 16
_LANES = 16
_NC = 2
_NS = 16
_NW = _NC * _NS
_BATCH = 16384
_RPW = _BATCH // _NW
_CHUNK = 128
_NCHUNK = _RPW // _CHUNK
_ACOL = _NUM_D
_BCOL = _NUM_D + _NUM_OPT * _NUM_D


def _irt_body(uid_hbm, qid_hbm, tab_hbm, out_hbm,
              uid_v, qid_v, u_b0, u_b1, q_b0, q_b1, o_v, sem0, sem1):
    wid = lax.axis_index("s") * _NC + lax.axis_index("c")

    pltpu.sync_copy(uid_hbm.at[wid], uid_v)
    pltpu.sync_copy(qid_hbm.at[wid], qid_v)

    u_b = (u_b0, u_b1)
    q_b = (q_b0, q_b1)
    sems = (sem0, sem1)

    def fire(j):
        p = j % 2
        return [
            pltpu.async_copy(tab_hbm.at[uid_v.at[j]], u_b[p], sems[p]),
            pltpu.async_copy(tab_hbm.at[qid_v.at[j]], q_b[p], sems[p]),
        ]

    lanes = lax.iota(jnp.int32, _LANES)

    def compute(j):
        p = j % 2
        for blk in range(_CHUNK // _LANES):
            rloc = lanes + blk * _LANES
            acc = [plsc.load_gather(q_b[p], [rloc, jnp.full((_LANES,), _BCOL + k, jnp.int32)])
                   for k in range(_NUM_OPT)]
            for d in range(_NUM_D):
                th_d = plsc.load_gather(u_b[p], [rloc, jnp.full((_LANES,), d, jnp.int32)])
                for k in range(_NUM_OPT):
                    a_kd = plsc.load_gather(
                        q_b[p], [rloc, jnp.full((_LANES,), _ACOL + d * _NUM_OPT + k, jnp.int32)])
                    acc[k] = acc[k] + a_kd * th_d
            for k in range(_NUM_OPT):
                flat = (rloc + j * _CHUNK) * _NUM_OPT + k
                plsc.store_scatter(
                    o_v,
                    [lax.shift_right_logical(flat, 7), lax.bitwise_and(flat, 127)],
                    acc[k])

    pending = fire(0)
    for j in range(_NCHUNK):
        nxt = fire(j + 1) if j + 1 < _NCHUNK else []
        for c in pending:
            c.wait()
        pending = nxt
        compute(j)

    nrow_o = _RPW * _NUM_OPT // 128
    pltpu.sync_copy(o_v, out_hbm.at[pl.ds(wid * nrow_o, nrow_o)])


_sc_call = functools.partial(
    pl.kernel,
    mesh=plsc.VectorSubcoreMesh(core_axis_name="c", subcore_axis_name="s"),
    compiler_params=pltpu.CompilerParams(
        needs_layout_passes=False, use_tc_tiling_on_sc=True),
    out_type=jax.ShapeDtypeStruct((_BATCH * _NUM_OPT // 128, 128), jnp.float32),
    scratch_types=[
        pltpu.VMEM((_NCHUNK, _CHUNK), jnp.int32),
        pltpu.VMEM((_NCHUNK, _CHUNK), jnp.int32),
        pltpu.VMEM((_CHUNK, 128), jnp.float32),
        pltpu.VMEM((_CHUNK, 128), jnp.float32),
        pltpu.VMEM((_CHUNK, 128), jnp.float32),
        pltpu.VMEM((_CHUNK, 128), jnp.float32),
        pltpu.VMEM((_RPW * _NUM_OPT // 128, 128), jnp.float32),
        pltpu.SemaphoreType.DMA,
        pltpu.SemaphoreType.DMA,
    ],
)(_irt_body)


def _pack_body(tht_ref, at_ref, bt_ref, out_ref):
    th = tht_ref[...]
    av = at_ref[...]
    bv = bt_ref[...]
    dn = (((0,), (0,)), ((), ()))
    tt = lax.dot_general(th, jnp.eye(_NUM_D, dtype=jnp.float32), dn,
                         preferred_element_type=jnp.float32)
    at = lax.dot_general(av, jnp.eye(_NUM_OPT * _NUM_D, dtype=jnp.float32), dn,
                         preferred_element_type=jnp.float32)
    bt = lax.dot_general(bv, jnp.eye(_NUM_OPT, dtype=jnp.float32), dn,
                         preferred_element_type=jnp.float32)
    z = jnp.zeros((128, 128 - _BCOL - _NUM_OPT), jnp.float32)
    out_ref[...] = jnp.concatenate([tt, at, bt, z], axis=1)


def _pack(tht, at2, bt, n):
    grid = (pl.cdiv(n, 128),)
    return pl.pallas_call(
        _pack_body,
        grid=grid,
        in_specs=[
            pl.BlockSpec((_NUM_D, 128), lambda i: (0, i)),
            pl.BlockSpec((_NUM_OPT * _NUM_D, 128), lambda i: (0, i)),
            pl.BlockSpec((_NUM_OPT, 128), lambda i: (0, i)),
        ],
        out_specs=pl.BlockSpec((128, 128), lambda i: (i, 0)),
        out_shape=jax.ShapeDtypeStruct((n, 128), jnp.float32),
    )(tht, at2, bt)


@jax.jit
def kernel(x, a, b, theta):
    uids = x[:, 0].astype(jnp.int32).reshape(_NW, _NCHUNK, _CHUNK)
    qids = x[:, 1].astype(jnp.int32).reshape(_NW, _NCHUNK, _CHUNK)
    n = theta.shape[0]
    tht = theta.T
    at2 = jnp.transpose(a, (2, 1, 0)).reshape(_NUM_OPT * _NUM_D, n)
    bt = b.T
    tab = _pack(tht, at2, bt, n)
    out = _sc_call(uids, qids, tab)
    return out.reshape(_BATCH, _NUM_OPT)


# speedup vs baseline: 3.4939x; 3.4939x over previous
import functools

import jax
import jax.numpy as jnp
from jax import lax
from jax.experimental import pallas as pl
from jax.experimental.pallas import tpu as pltpu
from jax.experimental.pallas import tpu_sc as plsc

_NUM_OPT = 4
_NUM_D = 16
_LANES = 16
_NC = 2
_NS = 16
_NW = _NC * _NS
_BATCH = 16384
_RPW = _BATCH // _NW
_CHUNK = 128
_NCHUNK = _RPW // _CHUNK
_ACOL = _NUM_D
_BCOL = _NUM_D + _NUM_OPT * _NUM_D


def _irt_body(uid_hbm, qid_hbm, tab_hbm, out_hbm,
              uid_v, qid_v, u_b0, u_b1, u_b2, q_b0, q_b1, q_b2, o_v,
              sem0, sem1, sem2):
    wid = lax.axis_index("s") * _NC + lax.axis_index("c")

    pltpu.sync_copy(uid_hbm.at[wid], uid_v)
    pltpu.sync_copy(qid_hbm.at[wid], qid_v)

    u_b = (u_b0, u_b1, u_b2)
    q_b = (q_b0, q_b1, q_b2)
    sems = (sem0, sem1, sem2)

    def fire(j):
        p = j % 3
        return [
            pltpu.async_copy(tab_hbm.at[uid_v.at[j]], u_b[p], sems[p]),
            pltpu.async_copy(tab_hbm.at[qid_v.at[j]], q_b[p], sems[p]),
        ]

    lanes = lax.iota(jnp.int32, _LANES)

    def compute(j):
        p = j % 3
        for blk in range(_CHUNK // _LANES):
            rloc = lanes + blk * _LANES
            acc = [plsc.load_gather(q_b[p], [rloc, jnp.full((_LANES,), _BCOL + k, jnp.int32)])
                   for k in range(_NUM_OPT)]
            for d in range(_NUM_D):
                th_d = plsc.load_gather(u_b[p], [rloc, jnp.full((_LANES,), d, jnp.int32)])
                for k in range(_NUM_OPT):
                    a_kd = plsc.load_gather(
                        q_b[p], [rloc, jnp.full((_LANES,), _ACOL + k * _NUM_D + d, jnp.int32)])
                    acc[k] = acc[k] + a_kd * th_d
            for k in range(_NUM_OPT):
                flat = (rloc + j * _CHUNK) * _NUM_OPT + k
                plsc.store_scatter(
                    o_v,
                    [lax.shift_right_logical(flat, 7), lax.bitwise_and(flat, 127)],
                    acc[k])

    inflight = [fire(0), fire(1)]
    for j in range(_NCHUNK):
        if j + 2 < _NCHUNK:
            inflight.append(fire(j + 2))
        for c in inflight.pop(0):
            c.wait()
        compute(j)

    nrow_o = _RPW * _NUM_OPT // 128
    pltpu.sync_copy(o_v, out_hbm.at[pl.ds(wid * nrow_o, nrow_o)])


_sc_call = functools.partial(
    pl.kernel,
    mesh=plsc.VectorSubcoreMesh(core_axis_name="c", subcore_axis_name="s"),
    compiler_params=pltpu.CompilerParams(
        needs_layout_passes=False, use_tc_tiling_on_sc=True),
    out_type=jax.ShapeDtypeStruct((_BATCH * _NUM_OPT // 128, 128), jnp.float32),
    scratch_types=[
        pltpu.VMEM((_NCHUNK, _CHUNK), jnp.int32),
        pltpu.VMEM((_NCHUNK, _CHUNK), jnp.int32),
        pltpu.VMEM((_CHUNK, 128), jnp.float32),
        pltpu.VMEM((_CHUNK, 128), jnp.float32),
        pltpu.VMEM((_CHUNK, 128), jnp.float32),
        pltpu.VMEM((_CHUNK, 128), jnp.float32),
        pltpu.VMEM((_CHUNK, 128), jnp.float32),
        pltpu.VMEM((_CHUNK, 128), jnp.float32),
        pltpu.VMEM((_RPW * _NUM_OPT // 128, 128), jnp.float32),
        pltpu.SemaphoreType.DMA,
        pltpu.SemaphoreType.DMA,
        pltpu.SemaphoreType.DMA,
    ],
)(_irt_body)


@jax.jit
def kernel(x, a, b, theta):
    uids = x[:, 0].astype(jnp.int32).reshape(_NW, _NCHUNK, _CHUNK)
    qids = x[:, 1].astype(jnp.int32).reshape(_NW, _NCHUNK, _CHUNK)
    n = theta.shape[0]
    tab = (jnp.pad(theta, ((0, 0), (0, 128 - _NUM_D)))
           + jnp.pad(a.reshape(n, _NUM_OPT * _NUM_D),
                     ((0, 0), (_ACOL, 128 - _BCOL)))
           + jnp.pad(b, ((0, 0), (_BCOL, 128 - _BCOL - _NUM_OPT))))
    out = _sc_call(uids, qids, tab)
    return out.reshape(_BATCH, _NUM_OPT)


# speedup vs baseline: 3.5207x; 1.0077x over previous
import functools

import jax
import jax.numpy as jnp
from jax import lax
from jax.experimental import pallas as pl
from jax.experimental.pallas import tpu as pltpu
from jax.experimental.pallas import tpu_sc as plsc

_NUM_OPT = 4
_NUM_D = 16
_LANES = 16
_NC = 2
_NS = 16
_NW = _NC * _NS
_BATCH = 16384
_RPW = _BATCH // _NW
_CHUNK = 128
_NCHUNK = _RPW // _CHUNK
_ACOL = _NUM_D
_BCOL = _NUM_D + _NUM_OPT * _NUM_D


def _irt_body(uid_hbm, qid_hbm, tab_hbm, out_hbm,
              uid_v, qid_v, u_b0, u_b1, q_b0, q_b1, o_v, sem0, sem1):
    wid = lax.axis_index("s") * _NC + lax.axis_index("c")

    pltpu.sync_copy(uid_hbm.at[wid], uid_v)
    pltpu.sync_copy(qid_hbm.at[wid], qid_v)

    u_b = (u_b0, u_b1)
    q_b = (q_b0, q_b1)
    sems = (sem0, sem1)

    def fire(j):
        p = j % 2
        return [
            pltpu.async_copy(tab_hbm.at[uid_v.at[j]], u_b[p], sems[p]),
            pltpu.async_copy(tab_hbm.at[qid_v.at[j]], q_b[p], sems[p]),
        ]

    lanes = lax.iota(jnp.int32, _LANES)

    def compute(j):
        p = j % 2
        for blk in range(_CHUNK // _LANES):
            rloc = lanes + blk * _LANES
            acc = [plsc.load_gather(q_b[p], [rloc, jnp.full((_LANES,), _BCOL + k, jnp.int32)])
                   for k in range(_NUM_OPT)]
            for d in range(_NUM_D):
                th_d = plsc.load_gather(u_b[p], [rloc, jnp.full((_LANES,), d, jnp.int32)])
                for k in range(_NUM_OPT):
                    a_kd = plsc.load_gather(
                        q_b[p], [rloc, jnp.full((_LANES,), _ACOL + k * _NUM_D + d, jnp.int32)])
                    acc[k] = acc[k] + a_kd * th_d
            for k in range(_NUM_OPT):
                flat = (rloc + j * _CHUNK) * _NUM_OPT + k
                plsc.store_scatter(
                    o_v,
                    [lax.shift_right_logical(flat, 7), lax.bitwise_and(flat, 127)],
                    acc[k])

    pending = fire(0)
    for j in range(_NCHUNK):
        nxt = fire(j + 1) if j + 1 < _NCHUNK else []
        for c in pending:
            c.wait()
        pending = nxt
        compute(j)

    nrow_o = _RPW * _NUM_OPT // 128
    pltpu.sync_copy(o_v, out_hbm.at[pl.ds(wid * nrow_o, nrow_o)])


_sc_call = functools.partial(
    pl.kernel,
    mesh=plsc.VectorSubcoreMesh(core_axis_name="c", subcore_axis_name="s"),
    compiler_params=pltpu.CompilerParams(
        needs_layout_passes=False, use_tc_tiling_on_sc=True),
    out_type=jax.ShapeDtypeStruct((_BATCH * _NUM_OPT // 128, 128), jnp.float32),
    scratch_types=[
        pltpu.VMEM((_NCHUNK, _CHUNK), jnp.int32),
        pltpu.VMEM((_NCHUNK, _CHUNK), jnp.int32),
        pltpu.VMEM((_CHUNK, 128), jnp.float32),
        pltpu.VMEM((_CHUNK, 128), jnp.float32),
        pltpu.VMEM((_CHUNK, 128), jnp.float32),
        pltpu.VMEM((_CHUNK, 128), jnp.float32),
        pltpu.VMEM((_RPW * _NUM_OPT // 128, 128), jnp.float32),
        pltpu.SemaphoreType.DMA,
        pltpu.SemaphoreType.DMA,
    ],
)(_irt_body)


@jax.jit
def kernel(x, a, b, theta):
    uids = x[:, 0].astype(jnp.int32).reshape(_NW, _NCHUNK, _CHUNK)
    qids = x[:, 1].astype(jnp.int32).reshape(_NW, _NCHUNK, _CHUNK)
    n = theta.shape[0]
    tab = (jnp.pad(theta, ((0, 0), (0, 128 - _NUM_D)))
           + jnp.pad(a.reshape(n, _NUM_OPT * _NUM_D),
                     ((0, 0), (_ACOL, 128 - _BCOL)))
           + jnp.pad(jnp.pad(b, ((0, 0), (0, 4))),
                     ((0, 0), (_BCOL, 128 - _BCOL - _NUM_OPT - 4))))
    out = _sc_call(uids, qids, tab)
    return out.reshape(_BATCH, _NUM_OPT)


# speedup vs baseline: 3.5223x; 1.0004x over previous
import functools

import jax
import jax.numpy as jnp
from jax import lax
from jax.experimental import pallas as pl
from jax.experimental.pallas import tpu as pltpu
from jax.experimental.pallas import tpu_sc as plsc

_NUM_OPT = 4
_NUM_D = 16
_LANES = 16
_NC = 2
_NS = 16
_NW = _NC * _NS
_BATCH = 16384
_RPW = _BATCH // _NW
_CHUNK = 128
_NCHUNK = _RPW // _CHUNK
_ACOL = _NUM_D
_BCOL = _NUM_D + _NUM_OPT * _NUM_D


def _irt_body(uid_hbm, qid_hbm, tab_hbm, out_hbm,
              uid_v, qid_v, u_b0, u_b1, q_b0, q_b1, o_v, sem0, sem1):
    wid = lax.axis_index("s") * _NC + lax.axis_index("c")

    pltpu.sync_copy(uid_hbm.at[wid], uid_v)
    pltpu.sync_copy(qid_hbm.at[wid], qid_v)

    u_b = (u_b0, u_b1)
    q_b = (q_b0, q_b1)
    sems = (sem0, sem1)

    def fire(j):
        p = j % 2
        return [
            pltpu.async_copy(tab_hbm.at[uid_v.at[j]], u_b[p], sems[p]),
            pltpu.async_copy(tab_hbm.at[qid_v.at[j]], q_b[p], sems[p]),
        ]

    lanes = lax.iota(jnp.int32, _LANES)

    def compute(j):
        p = j % 2
        for blk in range(_CHUNK // _LANES):
            rloc = lanes + blk * _LANES
            acc = [plsc.load_gather(q_b[p], [rloc, jnp.full((_LANES,), _BCOL + k, jnp.int32)])
                   for k in range(_NUM_OPT)]
            for d in range(_NUM_D):
                th_d = plsc.load_gather(u_b[p], [rloc, jnp.full((_LANES,), d, jnp.int32)])
                for k in range(_NUM_OPT):
                    a_kd = plsc.load_gather(
                        q_b[p], [rloc, jnp.full((_LANES,), _ACOL + k * _NUM_D + d, jnp.int32)])
                    acc[k] = acc[k] + a_kd * th_d
            for k in range(_NUM_OPT):
                flat = (rloc + j * _CHUNK) * _NUM_OPT + k
                plsc.store_scatter(
                    o_v,
                    [lax.shift_right_logical(flat, 7), lax.bitwise_and(flat, 127)],
                    acc[k])

    pending = fire(0)
    for j in range(_NCHUNK):
        nxt = fire(j + 1) if j + 1 < _NCHUNK else []
        for c in pending:
            c.wait()
        pending = nxt
        compute(j)

    nrow_o = _RPW * _NUM_OPT // 128
    pltpu.sync_copy(o_v, out_hbm.at[pl.ds(wid * nrow_o, nrow_o)])


_sc_call = functools.partial(
    pl.kernel,
    mesh=plsc.VectorSubcoreMesh(core_axis_name="c", subcore_axis_name="s"),
    compiler_params=pltpu.CompilerParams(
        needs_layout_passes=False, use_tc_tiling_on_sc=True),
    out_type=jax.ShapeDtypeStruct((_BATCH * _NUM_OPT // 128, 128), jnp.float32),
    scratch_types=[
        pltpu.VMEM((_NCHUNK, _CHUNK), jnp.int32),
        pltpu.VMEM((_NCHUNK, _CHUNK), jnp.int32),
        pltpu.VMEM((_CHUNK, 128), jnp.float32),
        pltpu.VMEM((_CHUNK, 128), jnp.float32),
        pltpu.VMEM((_CHUNK, 128), jnp.float32),
        pltpu.VMEM((_CHUNK, 128), jnp.float32),
        pltpu.VMEM((_RPW * _NUM_OPT // 128, 128), jnp.float32),
        pltpu.SemaphoreType.DMA,
        pltpu.SemaphoreType.DMA,
    ],
)(_irt_body)


@jax.jit
def kernel(x, a, b, theta):
    uids = x[:, 0].astype(jnp.int32).reshape(_NW, _NCHUNK, _CHUNK)
    qids = x[:, 1].astype(jnp.int32).reshape(_NW, _NCHUNK, _CHUNK)
    n = theta.shape[0]
    tab = (jnp.pad(theta, ((0, 0), (0, 128 - _NUM_D)))
           + jnp.pad(a.reshape(n, _NUM_OPT * _NUM_D),
                     ((0, 0), (_ACOL, 128 - _BCOL)))
           + jnp.pad(b, ((0, 0), (_BCOL, 128 - _BCOL - _NUM_OPT))))
    out = _sc_call(uids, qids, tab)
    return out.reshape(_BATCH, _NUM_OPT)
